# baseline (speedup 1.0000x reference)
import functools

import jax
import jax.numpy as jnp
from jax import lax
from jax.experimental import pallas as pl
from jax.experimental.pallas import tpu as pltpu
from jax.experimental.pallas import tpu_sc as plsc

_ENC_PREC = lax.Precision.DEFAULT
_SCORE_PREC = lax.Precision.DEFAULT
_ADAPT_PREC = lax.Precision.DEFAULT

B = 512
DS = 115
NK = 16384
DK = 3072
TOPK = 10
KT = 512
NT = NK // KT
SLOT = 16

NC, NS = 2, 16
NW = NC * NS
ROWS = B * TOPK
RPW = ROWS // NW
CH = 16
NCHUNK = RPW // CH


def _mm(x, w, prec=_ADAPT_PREC):
    return lax.dot_general(x, w, (((1,), (1,)), ((), ())),
                           precision=prec, preferred_element_type=jnp.float32)


def _ln(x, g, b):
    mu = jnp.mean(x, axis=-1, keepdims=True)
    var = jnp.mean(jnp.square(x - mu), axis=-1, keepdims=True)
    return (x - mu) / jnp.sqrt(var + 1e-5) * g + b


GRP = 8
CCH = 128
NCH = KT // CCH
CSLOT = 48


def _scores_topk_body(state_ref, rw1, rb1, rw2, rb2, rw3, rb3, rg, rbn,
                      dw1, db1, dw2, db2, dg, dbn, k_ref,
                      cv_ref, ci_ref, sf_ref, q_scr, v8_ref, i8_ref):
    t = pl.program_id(0)
    neg = jnp.float32(-jnp.inf)

    @pl.when(t == 0)
    def _():
        s0 = state_ref[...]
        h = jax.nn.relu(_mm(s0, rw1[...], _ENC_PREC) + rb1[...])
        h = jax.nn.relu(_mm(h, rw2[...], _ENC_PREC) + rb2[...])
        q = _ln(_mm(h, rw3[...], _ENC_PREC) + rb3[...], rg[...], rbn[...])
        qn = jnp.sqrt(jnp.sum(q * q, axis=1, keepdims=True))
        q_scr[...] = q / jnp.maximum(qn, 1e-12)
        d = jax.nn.relu(_mm(s0, dw1[...], _ENC_PREC) + db1[...])
        sf_ref[...] = _ln(_mm(d, dw2[...], _ENC_PREC) + db2[...],
                          dg[...], dbn[...])

    kt = k_ref[...]
    n = jnp.sqrt(jnp.sum(kt * kt, axis=1, keepdims=True))
    kn = kt / jnp.maximum(n, 1e-12)
    s = lax.dot_general(q_scr[...], kn, (((1,), (1,)), ((), ())),
                        precision=_SCORE_PREC,
                        preferred_element_type=jnp.float32)

    itc = lax.broadcasted_iota(jnp.int32, (B, CCH), 1)
    vs, ids = [], []
    for c in range(NCH):
        sc = s[:, c * CCH:(c + 1) * CCH]
        base = t * KT + c * CCH
        for _ in range(TOPK):
            m = jnp.max(sc, axis=1, keepdims=True)
            pos = jnp.min(jnp.where(sc == m, itc, CCH), axis=1, keepdims=True)
            vs.append(m)
            ids.append(pos + base)
            sc = jnp.where(itc == pos, neg, sc)
    pad = CSLOT - NCH * TOPK
    v8_ref[t % GRP] = jnp.concatenate(
        vs + [jnp.full((B, pad), neg, jnp.float32)], axis=1)
    i8_ref[t % GRP] = jnp.concatenate(
        ids + [jnp.zeros((B, pad), jnp.int32)], axis=1)

    @pl.when(t % GRP == GRP - 1)
    def _():
        cv_ref[...] = jnp.concatenate([v8_ref[j] for j in range(GRP)], axis=1)
        ci_ref[...] = jnp.concatenate([i8_ref[j] for j in range(GRP)], axis=1)


def _merge_topk_body(cv_ref, ci_ref, out_ref):
    v = cv_ref[...]
    gi = ci_ref[...]
    w = NT * CSLOT
    it2 = lax.broadcasted_iota(jnp.int32, (B, w), 1)
    outs = []
    for _ in range(TOPK):
        m = jnp.max(v, axis=1, keepdims=True)
        pos = jnp.min(jnp.where(v == m, it2, w), axis=1, keepdims=True)
        hit = it2 == pos
        outs.append(jnp.sum(jnp.where(hit, gi, 0), axis=1, keepdims=True))
        v = jnp.where(hit, jnp.float32(-jnp.inf), v)
    outs.append(jnp.zeros((B, SLOT - TOPK), jnp.int32))
    out_ref[...] = jnp.concatenate(outs, axis=1)


def _adapter_body(emb_ref, sf_ref, aw1, ab1, aw2, ab2, ag, abn,
                  wq_r, bq_r, wk_r, bk_r, wv_r, bv_r, wo_r, bo_r,
                  hw1, hb1, hw2, hb2, out_ref):
    bb = sf_ref.shape[0]
    e = emb_ref[...]
    a = jax.nn.relu(_mm(e, aw1[...]) + ab1[...])
    kn = _ln(_mm(a, aw2[...]) + ab2[...], ag[...], abn[...])
    sf = sf_ref[...]
    q = _mm(sf, wq_r[...]) + bq_r[...]
    kk = (_mm(kn, wk_r[...]) + bk_r[...]).reshape(bb, TOPK, 256)
    vv = (_mm(kn, wv_r[...]) + bv_r[...]).reshape(bb, TOPK, 256)
    ctxs = []
    for h in range(4):
        sl = slice(h * 64, (h + 1) * 64)
        qh = q[:, sl]
        kh = kk[:, :, sl]
        vh = vv[:, :, sl]
        att = jnp.sum(qh[:, None, :] * kh, axis=2) * 0.125
        att = att - jnp.max(att, axis=1, keepdims=True)
        wgt = jnp.exp(att)
        wgt = wgt / jnp.sum(wgt, axis=1, keepdims=True)
        ctxs.append(jnp.sum(wgt[:, :, None] * vh, axis=1))
    ctx = jnp.concatenate(ctxs, axis=1)
    ao = _mm(ctx, wo_r[...]) + bo_r[...]
    comb = jnp.concatenate([sf, ao], axis=1)
    hh = jax.nn.relu(_mm(comb, hw1[...]) + hb1[...])
    out_ref[...] = _mm(hh, hw2[...]) + hb2[...]


def _sc_gather(idx3, keys_bank):
    mesh = plsc.VectorSubcoreMesh(core_axis_name="c", subcore_axis_name="s",
                                  num_cores=NC, num_subcores=NS)

    @functools.partial(
        pl.kernel,
        out_type=jax.ShapeDtypeStruct((NW * NCHUNK, CH, DK), jnp.float32),
        mesh=mesh,
        scratch_types=[
            pltpu.VMEM((NCHUNK, CH), jnp.int32),
            pltpu.VMEM((2, CH, DK), jnp.float32),
            pltpu.SemaphoreType.DMA((2,)),
            pltpu.SemaphoreType.DMA((2,)),
        ],
    )
    def gather_k(idx_hbm, tab_hbm, out_hbm, idx_v, rows_v, gsem, ssem):
        wid = lax.axis_index("s") * NC + lax.axis_index("c")
        pltpu.sync_copy(idx_hbm.at[wid], idx_v)
        gh = [None] * NCHUNK
        sh = [None] * NCHUNK
        for c in range(min(2, NCHUNK)):
            gh[c] = pltpu.async_copy(tab_hbm.at[idx_v.at[c]],
                                     rows_v.at[c % 2], gsem.at[c % 2])
        for c in range(NCHUNK):
            gh[c].wait()
            sh[c] = pltpu.async_copy(rows_v.at[c % 2],
                                     out_hbm.at[wid * NCHUNK + c], ssem.at[c % 2])
            if c + 2 < NCHUNK:
                sh[c].wait()
                gh[c + 2] = pltpu.async_copy(tab_hbm.at[idx_v.at[c + 2]],
                                             rows_v.at[c % 2], gsem.at[c % 2])
        for c in range(max(NCHUNK - 2, 0), NCHUNK):
            sh[c].wait()

    return gather_k(idx3, keys_bank)


def kernel(state, keys_bank, re_w1, re_b1, re_w2, re_b2, re_w3, re_b3, re_g, re_bn,
           de_w1, de_b1, de_w2, de_b2, de_g, de_bn,
           ad_w1, ad_b1, ad_w2, ad_b2, ad_g, ad_bn,
           wq, bq, wk, bk, wv, bv, wo, bo,
           ah_w1, ah_b1, ah_w2, ah_b2):
    r2 = lambda x: x.reshape(1, -1)

    eargs = (state, re_w1, r2(re_b1), re_w2, r2(re_b2), re_w3, r2(re_b3),
             r2(re_g), r2(re_bn), de_w1, r2(de_b1), de_w2, r2(de_b2),
             r2(de_g), r2(de_bn))
    cand_v, cand_i, sf = pl.pallas_call(
        _scores_topk_body,
        grid=(NT,),
        in_specs=[pl.BlockSpec(a.shape, lambda t, n=a.ndim: (0,) * n)
                  for a in eargs] + [
            pl.BlockSpec((KT, DK), lambda t: (t, 0)),
        ],
        out_specs=(pl.BlockSpec((B, GRP * CSLOT), lambda t: (0, t // GRP)),
                   pl.BlockSpec((B, GRP * CSLOT), lambda t: (0, t // GRP)),
                   pl.BlockSpec((B, 256), lambda t: (0, 0))),
        out_shape=(jax.ShapeDtypeStruct((B, NT * CSLOT), jnp.float32),
                   jax.ShapeDtypeStruct((B, NT * CSLOT), jnp.int32),
                   jax.ShapeDtypeStruct((B, 256), jnp.float32)),
        scratch_shapes=[
            pltpu.VMEM((B, DK), jnp.float32),
            pltpu.VMEM((GRP, B, CSLOT), jnp.float32),
            pltpu.VMEM((GRP, B, CSLOT), jnp.int32),
        ],
        compiler_params=pltpu.CompilerParams(
            dimension_semantics=("arbitrary",)),
    )(*eargs, keys_bank)

    topk16 = pl.pallas_call(
        _merge_topk_body,
        out_shape=jax.ShapeDtypeStruct((B, SLOT), jnp.int32),
    )(cand_v, cand_i)

    idx = topk16[:, :TOPK].reshape(NW, NCHUNK, CH)
    emb = _sc_gather(idx, keys_bank).reshape(ROWS, DK)

    gb = 16
    bb = B // gb
    wargs = (ad_w1, r2(ad_b1), ad_w2, r2(ad_b2), r2(ad_g), r2(ad_bn),
             wq, r2(bq), wk, r2(bk), wv, r2(bv), wo, r2(bo),
             ah_w1, r2(ah_b1), ah_w2, r2(ah_b2))
    logits = pl.pallas_call(
        _adapter_body,
        grid=(gb,),
        in_specs=[
            pl.BlockSpec((bb * TOPK, DK), lambda i: (i, 0)),
            pl.BlockSpec((bb, 256), lambda i: (i, 0)),
        ] + [pl.BlockSpec(a.shape, lambda i, n=a.ndim: (0,) * n) for a in wargs],
        out_specs=pl.BlockSpec((bb, 23), lambda i: (i, 0)),
        out_shape=jax.ShapeDtypeStruct((B, 23), jnp.float32),
        compiler_params=pltpu.CompilerParams(
            dimension_semantics=("arbitrary",)),
    )(emb, sf, *wargs)
    return logits


# speedup vs baseline: 1.6647x; 1.6647x over previous
import functools

import jax
import jax.numpy as jnp
from jax import lax
from jax.experimental import pallas as pl
from jax.experimental.pallas import tpu as pltpu
from jax.experimental.pallas import tpu_sc as plsc

_ENC_PREC = lax.Precision.DEFAULT
_SCORE_PREC = lax.Precision.DEFAULT
_ADAPT_PREC = lax.Precision.DEFAULT

B = 512
DS = 115
NK = 16384
DK = 3072
TOPK = 10
KT = 1024
NT = NK // KT
SLOT = 16

NC, NS = 2, 16
NW = NC * NS
ROWS = B * TOPK
RPW = ROWS // NW
CH = 16
NCHUNK = RPW // CH


def _mm(x, w, prec=_ADAPT_PREC):
    return lax.dot_general(x, w, (((1,), (1,)), ((), ())),
                           precision=prec, preferred_element_type=jnp.float32)


def _ln(x, g, b):
    mu = jnp.mean(x, axis=-1, keepdims=True)
    var = jnp.mean(jnp.square(x - mu), axis=-1, keepdims=True)
    return (x - mu) / jnp.sqrt(var + 1e-5) * g + b


GRP = 8
CSLOT = 16


def _scores_topk_body(state_ref, rw1, rb1, rw2, rb2, rw3, rb3, rg, rbn,
                      dw1, db1, dw2, db2, dg, dbn, k_ref,
                      cv_ref, ci_ref, sf_ref, q_scr, v8_ref, i8_ref):
    t = pl.program_id(0)
    neg = jnp.float32(-jnp.inf)

    @pl.when(t == 0)
    def _():
        s0 = state_ref[...]
        h = jax.nn.relu(_mm(s0, rw1[...], _ENC_PREC) + rb1[...])
        h = jax.nn.relu(_mm(h, rw2[...], _ENC_PREC) + rb2[...])
        q = _ln(_mm(h, rw3[...], _ENC_PREC) + rb3[...], rg[...], rbn[...])
        qn = jnp.sqrt(jnp.sum(q * q, axis=1, keepdims=True))
        q_scr[...] = q / jnp.maximum(qn, 1e-12)
        d = jax.nn.relu(_mm(s0, dw1[...], _ENC_PREC) + db1[...])
        sf_ref[...] = _ln(_mm(d, dw2[...], _ENC_PREC) + db2[...],
                          dg[...], dbn[...])

    kt = k_ref[...]
    n = jnp.sqrt(jnp.sum(kt * kt, axis=1, keepdims=True))
    kn = kt / jnp.maximum(n, 1e-12)
    s = lax.dot_general(q_scr[...], kn, (((1,), (1,)), ((), ())),
                        precision=_SCORE_PREC,
                        preferred_element_type=jnp.float32)

    it = lax.broadcasted_iota(jnp.int32, (B, KT), 1)
    vs, ids = [], []
    for _ in range(TOPK):
        m = jnp.max(s, axis=1, keepdims=True)
        pos = jnp.min(jnp.where(s == m, it, KT), axis=1, keepdims=True)
        vs.append(m)
        ids.append(pos + t * KT)
        s = jnp.where(it == pos, neg, s)
    pad = CSLOT - TOPK
    v8_ref[t % GRP] = jnp.concatenate(
        vs + [jnp.full((B, pad), neg, jnp.float32)], axis=1)
    i8_ref[t % GRP] = jnp.concatenate(
        ids + [jnp.zeros((B, pad), jnp.int32)], axis=1)

    @pl.when(t % GRP == GRP - 1)
    def _():
        cv_ref[...] = jnp.concatenate([v8_ref[j] for j in range(GRP)], axis=1)
        ci_ref[...] = jnp.concatenate([i8_ref[j] for j in range(GRP)], axis=1)


def _merge_topk_body(cv_ref, ci_ref, out_ref):
    v = cv_ref[...]
    gi = ci_ref[...]
    w = NT * CSLOT
    it2 = lax.broadcasted_iota(jnp.int32, (B, w), 1)
    outs = []
    for _ in range(TOPK):
        m = jnp.max(v, axis=1, keepdims=True)
        pos = jnp.min(jnp.where(v == m, it2, w), axis=1, keepdims=True)
        hit = it2 == pos
        outs.append(jnp.sum(jnp.where(hit, gi, 0), axis=1, keepdims=True))
        v = jnp.where(hit, jnp.float32(-jnp.inf), v)
    outs.append(jnp.zeros((B, SLOT - TOPK), jnp.int32))
    out_ref[...] = jnp.concatenate(outs, axis=1)


def _adapter_body(emb_ref, sf_ref, aw1, ab1, aw2, ab2, ag, abn,
                  wq_r, bq_r, wk_r, bk_r, wv_r, bv_r, wo_r, bo_r,
                  hw1, hb1, hw2, hb2, out_ref):
    bb = sf_ref.shape[0]
    e = emb_ref[...]
    a = jax.nn.relu(_mm(e, aw1[...]) + ab1[...])
    kn = _ln(_mm(a, aw2[...]) + ab2[...], ag[...], abn[...])
    sf = sf_ref[...]
    q = _mm(sf, wq_r[...]) + bq_r[...]
    kk = (_mm(kn, wk_r[...]) + bk_r[...]).reshape(bb, TOPK, 256)
    vv = (_mm(kn, wv_r[...]) + bv_r[...]).reshape(bb, TOPK, 256)
    ctxs = []
    for h in range(4):
        sl = slice(h * 64, (h + 1) * 64)
        qh = q[:, sl]
        kh = kk[:, :, sl]
        vh = vv[:, :, sl]
        att = jnp.sum(qh[:, None, :] * kh, axis=2) * 0.125
        att = att - jnp.max(att, axis=1, keepdims=True)
        wgt = jnp.exp(att)
        wgt = wgt / jnp.sum(wgt, axis=1, keepdims=True)
        ctxs.append(jnp.sum(wgt[:, :, None] * vh, axis=1))
    ctx = jnp.concatenate(ctxs, axis=1)
    ao = _mm(ctx, wo_r[...]) + bo_r[...]
    comb = jnp.concatenate([sf, ao], axis=1)
    hh = jax.nn.relu(_mm(comb, hw1[...]) + hb1[...])
    out_ref[...] = _mm(hh, hw2[...]) + hb2[...]


def _sc_gather(idx3, keys_bank):
    mesh = plsc.VectorSubcoreMesh(core_axis_name="c", subcore_axis_name="s",
                                  num_cores=NC, num_subcores=NS)

    @functools.partial(
        pl.kernel,
        out_type=jax.ShapeDtypeStruct((NW * NCHUNK, CH, DK), jnp.float32),
        mesh=mesh,
        scratch_types=[
            pltpu.VMEM((NCHUNK, CH), jnp.int32),
            pltpu.VMEM((2, CH, DK), jnp.float32),
            pltpu.SemaphoreType.DMA((2,)),
            pltpu.SemaphoreType.DMA((2,)),
        ],
    )
    def gather_k(idx_hbm, tab_hbm, out_hbm, idx_v, rows_v, gsem, ssem):
        wid = lax.axis_index("s") * NC + lax.axis_index("c")
        pltpu.sync_copy(idx_hbm.at[wid], idx_v)
        gh = [None] * NCHUNK
        sh = [None] * NCHUNK
        for c in range(min(2, NCHUNK)):
            gh[c] = pltpu.async_copy(tab_hbm.at[idx_v.at[c]],
                                     rows_v.at[c % 2], gsem.at[c % 2])
        for c in range(NCHUNK):
            gh[c].wait()
            sh[c] = pltpu.async_copy(rows_v.at[c % 2],
                                     out_hbm.at[wid * NCHUNK + c], ssem.at[c % 2])
            if c + 2 < NCHUNK:
                sh[c].wait()
                gh[c + 2] = pltpu.async_copy(tab_hbm.at[idx_v.at[c + 2]],
                                             rows_v.at[c % 2], gsem.at[c % 2])
        for c in range(max(NCHUNK - 2, 0), NCHUNK):
            sh[c].wait()

    return gather_k(idx3, keys_bank)


def kernel(state, keys_bank, re_w1, re_b1, re_w2, re_b2, re_w3, re_b3, re_g, re_bn,
           de_w1, de_b1, de_w2, de_b2, de_g, de_bn,
           ad_w1, ad_b1, ad_w2, ad_b2, ad_g, ad_bn,
           wq, bq, wk, bk, wv, bv, wo, bo,
           ah_w1, ah_b1, ah_w2, ah_b2):
    r2 = lambda x: x.reshape(1, -1)

    eargs = (state, re_w1, r2(re_b1), re_w2, r2(re_b2), re_w3, r2(re_b3),
             r2(re_g), r2(re_bn), de_w1, r2(de_b1), de_w2, r2(de_b2),
             r2(de_g), r2(de_bn))
    cand_v, cand_i, sf = pl.pallas_call(
        _scores_topk_body,
        grid=(NT,),
        in_specs=[pl.BlockSpec(a.shape, lambda t, n=a.ndim: (0,) * n)
                  for a in eargs] + [
            pl.BlockSpec((KT, DK), lambda t: (t, 0)),
        ],
        out_specs=(pl.BlockSpec((B, GRP * CSLOT), lambda t: (0, t // GRP)),
                   pl.BlockSpec((B, GRP * CSLOT), lambda t: (0, t // GRP)),
                   pl.BlockSpec((B, 256), lambda t: (0, 0))),
        out_shape=(jax.ShapeDtypeStruct((B, NT * CSLOT), jnp.float32),
                   jax.ShapeDtypeStruct((B, NT * CSLOT), jnp.int32),
                   jax.ShapeDtypeStruct((B, 256), jnp.float32)),
        scratch_shapes=[
            pltpu.VMEM((B, DK), jnp.float32),
            pltpu.VMEM((GRP, B, CSLOT), jnp.float32),
            pltpu.VMEM((GRP, B, CSLOT), jnp.int32),
        ],
        compiler_params=pltpu.CompilerParams(
            dimension_semantics=("arbitrary",)),
    )(*eargs, keys_bank)

    topk16 = pl.pallas_call(
        _merge_topk_body,
        out_shape=jax.ShapeDtypeStruct((B, SLOT), jnp.int32),
    )(cand_v, cand_i)

    idx = topk16[:, :TOPK].reshape(NW, NCHUNK, CH)
    emb = _sc_gather(idx, keys_bank).reshape(ROWS, DK)

    gb = 8
    bb = B // gb
    wargs = (ad_w1, r2(ad_b1), ad_w2, r2(ad_b2), r2(ad_g), r2(ad_bn),
             wq, r2(bq), wk, r2(bk), wv, r2(bv), wo, r2(bo),
             ah_w1, r2(ah_b1), ah_w2, r2(ah_b2))
    logits = pl.pallas_call(
        _adapter_body,
        grid=(gb,),
        in_specs=[
            pl.BlockSpec((bb * TOPK, DK), lambda i: (i, 0)),
            pl.BlockSpec((bb, 256), lambda i: (i, 0)),
        ] + [pl.BlockSpec(a.shape, lambda i, n=a.ndim: (0,) * n) for a in wargs],
        out_specs=pl.BlockSpec((bb, 23), lambda i: (i, 0)),
        out_shape=jax.ShapeDtypeStruct((B, 23), jnp.float32),
        compiler_params=pltpu.CompilerParams(
            dimension_semantics=("arbitrary",)),
    )(emb, sf, *wargs)
    return logits


# speedup vs baseline: 1.7430x; 1.0470x over previous
import functools

import jax
import jax.numpy as jnp
from jax import lax
from jax.experimental import pallas as pl
from jax.experimental.pallas import tpu as pltpu
from jax.experimental.pallas import tpu_sc as plsc

_ENC_PREC = lax.Precision.DEFAULT
_SCORE_PREC = lax.Precision.DEFAULT
_ADAPT_PREC = lax.Precision.DEFAULT

B = 512
DS = 115
NK = 16384
DK = 3072
TOPK = 10
KT = 1024
NT = NK // KT
SLOT = 16

NC, NS = 2, 16
NW = NC * NS
ROWS = B * TOPK
RPW = ROWS // NW
CH = 16
NCHUNK = RPW // CH


def _mm(x, w, prec=_ADAPT_PREC):
    return lax.dot_general(x, w, (((1,), (1,)), ((), ())),
                           precision=prec, preferred_element_type=jnp.float32)


def _ln(x, g, b):
    mu = jnp.mean(x, axis=-1, keepdims=True)
    var = jnp.mean(jnp.square(x - mu), axis=-1, keepdims=True)
    return (x - mu) / jnp.sqrt(var + 1e-5) * g + b


GRP = 8
CSLOT = 16


def _scores_topk_body(state_ref, rw1, rb1, rw2, rb2, rw3, rb3, rg, rbn,
                      dw1, db1, dw2, db2, dg, dbn, k_ref,
                      cv_ref, ci_ref, sf_ref, q_scr, v8_ref, i8_ref):
    t = pl.program_id(0)
    neg = jnp.float32(-jnp.inf)

    @pl.when(t == 0)
    def _():
        s0 = state_ref[...]
        h = jax.nn.relu(_mm(s0, rw1[...], _ENC_PREC) + rb1[...])
        h = jax.nn.relu(_mm(h, rw2[...], _ENC_PREC) + rb2[...])
        q = _ln(_mm(h, rw3[...], _ENC_PREC) + rb3[...], rg[...], rbn[...])
        qn = jnp.sqrt(jnp.sum(q * q, axis=1, keepdims=True))
        q_scr[...] = q / jnp.maximum(qn, 1e-12)
        d = jax.nn.relu(_mm(s0, dw1[...], _ENC_PREC) + db1[...])
        sf_ref[...] = _ln(_mm(d, dw2[...], _ENC_PREC) + db2[...],
                          dg[...], dbn[...])

    kt = k_ref[...]
    n = jnp.sqrt(jnp.sum(kt * kt, axis=1, keepdims=True))
    kn = kt / jnp.maximum(n, 1e-12)
    s = lax.dot_general(q_scr[...], kn, (((1,), (1,)), ((), ())),
                        precision=_SCORE_PREC,
                        preferred_element_type=jnp.float32)

    it = lax.broadcasted_iota(jnp.int32, (B, KT), 1)
    vs, ids = [], []
    for _ in range(TOPK):
        m = jnp.max(s, axis=1, keepdims=True)
        pos = jnp.min(jnp.where(s == m, it, KT), axis=1, keepdims=True)
        vs.append(m)
        ids.append(pos + t * KT)
        s = jnp.where(it == pos, neg, s)
    pad = CSLOT - TOPK
    v8_ref[t % GRP] = jnp.concatenate(
        vs + [jnp.full((B, pad), neg, jnp.float32)], axis=1)
    i8_ref[t % GRP] = jnp.concatenate(
        ids + [jnp.zeros((B, pad), jnp.int32)], axis=1)

    @pl.when(t % GRP == GRP - 1)
    def _():
        cv_ref[...] = jnp.concatenate([v8_ref[j] for j in range(GRP)], axis=1)
        ci_ref[...] = jnp.concatenate([i8_ref[j] for j in range(GRP)], axis=1)


def _merge_topk_body(cv_ref, ci_ref, out_ref):
    v = cv_ref[...]
    gi = ci_ref[...]
    w = NT * CSLOT
    it2 = lax.broadcasted_iota(jnp.int32, (B, w), 1)
    outs = []
    for _ in range(TOPK):
        m = jnp.max(v, axis=1, keepdims=True)
        pos = jnp.min(jnp.where(v == m, it2, w), axis=1, keepdims=True)
        hit = it2 == pos
        outs.append(jnp.sum(jnp.where(hit, gi, 0), axis=1, keepdims=True))
        v = jnp.where(hit, jnp.float32(-jnp.inf), v)
    outs.append(jnp.zeros((B, SLOT - TOPK), jnp.int32))
    out_ref[...] = jnp.concatenate(outs, axis=1)


def _adapter_body(emb_ref, sf_ref, aw1, ab1, aw2, ab2, ag, abn,
                  wq_r, bq_r, wk_r, bk_r, wv_r, bv_r, wo_r, bo_r,
                  hw1, hb1, hw2, hb2, out_ref):
    bb = sf_ref.shape[0]
    e = emb_ref[...]
    a = jax.nn.relu(_mm(e, aw1[...]) + ab1[...])
    kn = _ln(_mm(a, aw2[...]) + ab2[...], ag[...], abn[...])
    sf = sf_ref[...]
    q = _mm(sf, wq_r[...]) + bq_r[...]
    kk = (_mm(kn, wk_r[...]) + bk_r[...]).reshape(bb, TOPK, 256)
    vv = (_mm(kn, wv_r[...]) + bv_r[...]).reshape(bb, TOPK, 256)
    ctxs = []
    for h in range(4):
        sl = slice(h * 64, (h + 1) * 64)
        qh = q[:, sl]
        kh = kk[:, :, sl]
        vh = vv[:, :, sl]
        att = jnp.sum(qh[:, None, :] * kh, axis=2) * 0.125
        att = att - jnp.max(att, axis=1, keepdims=True)
        wgt = jnp.exp(att)
        wgt = wgt / jnp.sum(wgt, axis=1, keepdims=True)
        ctxs.append(jnp.sum(wgt[:, :, None] * vh, axis=1))
    ctx = jnp.concatenate(ctxs, axis=1)
    ao = _mm(ctx, wo_r[...]) + bo_r[...]
    comb = jnp.concatenate([sf, ao], axis=1)
    hh = jax.nn.relu(_mm(comb, hw1[...]) + hb1[...])
    out_ref[...] = _mm(hh, hw2[...]) + hb2[...]


def _sc_gather(idx3, keys_bank):
    nchunk = idx3.shape[1]
    mesh = plsc.VectorSubcoreMesh(core_axis_name="c", subcore_axis_name="s",
                                  num_cores=NC, num_subcores=NS)

    @functools.partial(
        pl.kernel,
        out_type=jax.ShapeDtypeStruct((NW * nchunk, CH, DK), jnp.float32),
        mesh=mesh,
        scratch_types=[
            pltpu.VMEM((nchunk, CH), jnp.int32),
            pltpu.VMEM((2, CH, DK), jnp.float32),
            pltpu.SemaphoreType.DMA((2,)),
            pltpu.SemaphoreType.DMA((2,)),
        ],
    )
    def gather_k(idx_hbm, tab_hbm, out_hbm, idx_v, rows_v, gsem, ssem):
        wid = lax.axis_index("s") * NC + lax.axis_index("c")
        pltpu.sync_copy(idx_hbm.at[wid], idx_v)
        gh = [None] * nchunk
        sh = [None] * nchunk
        for c in range(min(2, nchunk)):
            gh[c] = pltpu.async_copy(tab_hbm.at[idx_v.at[c]],
                                     rows_v.at[c % 2], gsem.at[c % 2])
        for c in range(nchunk):
            gh[c].wait()
            sh[c] = pltpu.async_copy(rows_v.at[c % 2],
                                     out_hbm.at[wid * nchunk + c], ssem.at[c % 2])
            if c + 2 < nchunk:
                sh[c].wait()
                gh[c + 2] = pltpu.async_copy(tab_hbm.at[idx_v.at[c + 2]],
                                             rows_v.at[c % 2], gsem.at[c % 2])
        for c in range(max(nchunk - 2, 0), nchunk):
            sh[c].wait()

    return gather_k(idx3, keys_bank)


def kernel(state, keys_bank, re_w1, re_b1, re_w2, re_b2, re_w3, re_b3, re_g, re_bn,
           de_w1, de_b1, de_w2, de_b2, de_g, de_bn,
           ad_w1, ad_b1, ad_w2, ad_b2, ad_g, ad_bn,
           wq, bq, wk, bk, wv, bv, wo, bo,
           ah_w1, ah_b1, ah_w2, ah_b2):
    r2 = lambda x: x.reshape(1, -1)

    eargs = (state, re_w1, r2(re_b1), re_w2, r2(re_b2), re_w3, r2(re_b3),
             r2(re_g), r2(re_bn), de_w1, r2(de_b1), de_w2, r2(de_b2),
             r2(de_g), r2(de_bn))
    cand_v, cand_i, sf = pl.pallas_call(
        _scores_topk_body,
        grid=(NT,),
        in_specs=[pl.BlockSpec(a.shape, lambda t, n=a.ndim: (0,) * n)
                  for a in eargs] + [
            pl.BlockSpec((KT, DK), lambda t: (t, 0)),
        ],
        out_specs=(pl.BlockSpec((B, GRP * CSLOT), lambda t: (0, t // GRP)),
                   pl.BlockSpec((B, GRP * CSLOT), lambda t: (0, t // GRP)),
                   pl.BlockSpec((B, 256), lambda t: (0, 0))),
        out_shape=(jax.ShapeDtypeStruct((B, NT * CSLOT), jnp.float32),
                   jax.ShapeDtypeStruct((B, NT * CSLOT), jnp.int32),
                   jax.ShapeDtypeStruct((B, 256), jnp.float32)),
        scratch_shapes=[
            pltpu.VMEM((B, DK), jnp.float32),
            pltpu.VMEM((GRP, B, CSLOT), jnp.float32),
            pltpu.VMEM((GRP, B, CSLOT), jnp.int32),
        ],
        compiler_params=pltpu.CompilerParams(
            dimension_semantics=("arbitrary",)),
    )(*eargs, keys_bank)

    topk16 = pl.pallas_call(
        _merge_topk_body,
        out_shape=jax.ShapeDtypeStruct((B, SLOT), jnp.int32),
    )(cand_v, cand_i)

    wargs = (ad_w1, r2(ad_b1), ad_w2, r2(ad_b2), r2(ad_g), r2(ad_bn),
             wq, r2(bq), wk, r2(bk), wv, r2(bv), wo, r2(bo),
             ah_w1, r2(ah_b1), ah_w2, r2(ah_b2))

    def adapter_call(embh, sfh):
        nb = sfh.shape[0]
        bb = 64
        return pl.pallas_call(
            _adapter_body,
            grid=(nb // bb,),
            in_specs=[
                pl.BlockSpec((bb * TOPK, DK), lambda i: (i, 0)),
                pl.BlockSpec((bb, 256), lambda i: (i, 0)),
            ] + [pl.BlockSpec(a.shape, lambda i, n=a.ndim: (0,) * n)
                 for a in wargs],
            out_specs=pl.BlockSpec((bb, 23), lambda i: (i, 0)),
            out_shape=jax.ShapeDtypeStruct((nb, 23), jnp.float32),
            compiler_params=pltpu.CompilerParams(
                dimension_semantics=("arbitrary",)),
        )(embh, sfh, *wargs)

    HB = B // 2
    embs = []
    for h in range(2):
        idxh = topk16[h * HB:(h + 1) * HB, :TOPK].reshape(NW, -1, CH)
        embs.append(_sc_gather(idxh, keys_bank).reshape(HB * TOPK, DK))
    logits = jnp.concatenate(
        [adapter_call(embs[h], sf[h * HB:(h + 1) * HB]) for h in range(2)],
        axis=0)
    return logits
